# baseline (speedup 1.0000x reference)
import functools

import jax
import jax.numpy as jnp
from jax import lax
from jax.experimental import pallas as pl
from jax.experimental.pallas import tpu as pltpu
from jax.experimental.pallas import tpu_sc as plsc

D_HID = 1024
N_POS = 4096
TRAIN_H = 64
TRAIN_W = 64
N_SEL = TRAIN_H * TRAIN_W

_NUM_CORES = 2
_NUM_SUBCORES = 16
_NW = _NUM_CORES * _NUM_SUBCORES
_ROWS_PER_W = N_SEL // _NW
_CHUNK = 32
_NCHUNK = _ROWS_PER_W // _CHUNK


def _sc_gather(table, idx3):
    mesh = plsc.VectorSubcoreMesh(core_axis_name="c", subcore_axis_name="s")

    @functools.partial(
        pl.kernel,
        out_type=jax.ShapeDtypeStruct((N_SEL, D_HID), jnp.float32),
        mesh=mesh,
        scratch_types=[
            pltpu.VMEM((_NCHUNK, _CHUNK), jnp.int32),
            pltpu.VMEM((_CHUNK, D_HID), jnp.float32),
            pltpu.VMEM((_CHUNK, D_HID), jnp.float32),
            pltpu.SemaphoreType.DMA,
            pltpu.SemaphoreType.DMA,
        ],
    )
    def gather_kernel(table_hbm, idx_hbm, out_hbm, idx_v, buf0, buf1, sem0, sem1):
        wid = lax.axis_index("s") * _NUM_CORES + lax.axis_index("c")
        base = wid * _ROWS_PER_W
        pltpu.sync_copy(idx_hbm.at[wid], idx_v)
        bufs = (buf0, buf1)
        sems = (sem0, sem1)
        copies = [None] * _NCHUNK
        copies[0] = pltpu.async_copy(
            table_hbm.at[idx_v.at[0]], bufs[0], sems[0])
        for k in range(_NCHUNK):
            if k + 1 < _NCHUNK:
                copies[k + 1] = pltpu.async_copy(
                    table_hbm.at[idx_v.at[k + 1]],
                    bufs[(k + 1) % 2], sems[(k + 1) % 2])
            copies[k].wait()
            pltpu.sync_copy(bufs[k % 2],
                            out_hbm.at[pl.ds(base + k * _CHUNK, _CHUNK)])

    return gather_kernel(table, idx3)


_ROW_BLK = 256


def _tc_add_body(x_ref, pe_ref, o_ref):
    o_ref[...] = x_ref[...] + pe_ref[...][None, :, :]


def _tc_add(x, pe):
    b, n, d = x.shape
    nrb = n // _ROW_BLK
    return pl.pallas_call(
        _tc_add_body,
        grid=(nrb,),
        in_specs=[
            pl.BlockSpec((b, _ROW_BLK, d), lambda r: (0, r, 0)),
            pl.BlockSpec((_ROW_BLK, d), lambda r: (r, 0)),
        ],
        out_specs=pl.BlockSpec((b, _ROW_BLK, d), lambda r: (0, r, 0)),
        out_shape=jax.ShapeDtypeStruct(x.shape, x.dtype),
    )(x, pe)


def _sc_fused(x, table, idx3):
    mesh = plsc.VectorSubcoreMesh(core_axis_name="c", subcore_axis_name="s")
    nb = x.shape[0]
    nsteps = _NCHUNK * nb

    @functools.partial(
        pl.kernel,
        out_type=jax.ShapeDtypeStruct(x.shape, jnp.float32),
        mesh=mesh,
        scratch_types=[
            pltpu.VMEM((_NCHUNK, _CHUNK), jnp.int32),
            pltpu.VMEM((_CHUNK, D_HID), jnp.float32),
            pltpu.VMEM((_CHUNK, D_HID), jnp.float32),
            pltpu.VMEM((_CHUNK, D_HID), jnp.float32),
            pltpu.SemaphoreType.DMA,
            pltpu.SemaphoreType.DMA,
            pltpu.SemaphoreType.DMA,
            pltpu.SemaphoreType.DMA,
            pltpu.SemaphoreType.DMA,
        ],
    )
    def fused(x_hbm, table_hbm, idx_hbm, out_hbm,
              idx_v, rows, xb0, xb1, gsem, is0, is1, os0, os1):
        wid = lax.axis_index("s") * _NUM_CORES + lax.axis_index("c")
        base = wid * _ROWS_PER_W
        pltpu.sync_copy(idx_hbm.at[wid], idx_v)
        xbufs, isems, osems = (xb0, xb1), (is0, is1), (os0, os1)

        def fill(k, b, par):
            row0 = base + k * _CHUNK
            return pltpu.async_copy(
                x_hbm.at[b, pl.ds(row0, _CHUNK)], xbufs[par], isems[par])

        def add_inplace(xb):
            def body(r, carry):
                for j in range(D_HID // 16):
                    sl = pl.ds(j * 16, 16)
                    xb[r, sl] = xb[r, sl] + rows[r, sl]
                return carry
            lax.fori_loop(0, _CHUNK, body, 0)

        gh = pltpu.async_copy(table_hbm.at[idx_v.at[0]], rows, gsem)
        fills = [fill(0, 0, 0), None]
        drains = [None, None]
        for i in range(nsteps):
            k, b = divmod(i, nb)
            par = i % 2
            if i + 1 < nsteps:
                k2, b2 = divmod(i + 1, nb)
                npar = (i + 1) % 2
                if drains[npar] is not None:
                    drains[npar].wait()
                    drains[npar] = None
                fills[npar] = fill(k2, b2, npar)
            if b == 0:
                gh.wait()
            fills[par].wait()
            add_inplace(xbufs[par])
            if b == nb - 1 and k + 1 < _NCHUNK:
                gh = pltpu.async_copy(
                    table_hbm.at[idx_v.at[k + 1]], rows, gsem)
            row0 = base + k * _CHUNK
            drains[par] = pltpu.async_copy(
                xbufs[par], out_hbm.at[b, pl.ds(row0, _CHUNK)], osems[par])
        for d in drains:
            if d is not None:
                d.wait()

    return fused(x, table, idx3)


def kernel(x, pos_table, hash_index):
    sel = hash_index[:TRAIN_H, :TRAIN_W].reshape(-1).astype(jnp.int32)
    idx3 = sel.reshape(_NW, _NCHUNK, _CHUNK)
    table = pos_table.reshape(N_POS, D_HID)
    return _sc_fused(x, table, idx3)
